# Initial kernel scaffold; baseline (speedup 1.0000x reference)
import jax
import jax.numpy as jnp
from jax.experimental import pallas as pl


def kernel(points_xyz, points_features, W1, b1, Wg, bg, Wphi, bphi, Wpsi, bpsi, Wa, ba, Wd1, bd1, Wd2, bd2, W2, b2):
    raise NotImplementedError("write your pallas kernel here")


# speedup vs baseline: 13.3921x; 13.3921x over previous
import functools

import jax
import jax.numpy as jnp
from jax import lax
from jax.experimental import pallas as pl
from jax.experimental.pallas import tpu as pltpu
from jax.experimental.pallas import tpu_sc as plsc

_B, _N, _D, _K = 2, 4096, 128, 16
_TOT = _B * _N * _K

_RP = 512
_R2 = 256
_R4 = 128
_RK = _R4 * _K


def _k1_body(f_ref, xyz_ref, W1_ref, b1_ref, Wa_ref, bA_ref, Wpg_ref, bS_ref,
             Wpp_ref, bP_ref, T_ref, P_ref):
    f = f_ref[0]
    out = jnp.dot(f, W1_ref[...], preferred_element_type=jnp.float32) + b1_ref[...]
    T_ref[0, :, 0:128] = jnp.dot(out, Wpg_ref[...], preferred_element_type=jnp.float32) + bS_ref[...]
    T_ref[0, :, 128:256] = jnp.dot(out, Wa_ref[...], preferred_element_type=jnp.float32) + bA_ref[...]
    T_ref[0, :, 256:384] = xyz_ref[0]
    P_ref[0] = jnp.dot(out, Wpp_ref[...], preferred_element_type=jnp.float32) + bP_ref[...]


def _project(feat, xyzp128, W1, b1, Wa, bA, Wpg, bS, Wpp, bP):
    wspec = pl.BlockSpec((128, 128), lambda b, i: (0, 0))
    bspec = pl.BlockSpec((1, 128), lambda b, i: (0, 0))
    return pl.pallas_call(
        _k1_body,
        grid=(_B, _N // _RP),
        in_specs=[
            pl.BlockSpec((1, _RP, _D), lambda b, i: (b, i, 0)),
            pl.BlockSpec((1, _RP, _D), lambda b, i: (b, i, 0)),
            wspec, bspec, wspec, bspec, wspec, bspec, wspec, bspec,
        ],
        out_specs=[
            pl.BlockSpec((1, _RP, 384), lambda b, i: (b, i, 0)),
            pl.BlockSpec((1, _RP, _D), lambda b, i: (b, i, 0)),
        ],
        out_shape=[
            jax.ShapeDtypeStruct((_B, _N, 384), jnp.float32),
            jax.ShapeDtypeStruct((_B, _N, _D), jnp.float32),
        ],
    )(feat, xyzp128, W1, b1, Wa, bA, Wpg, bS, Wpp, bP)


def _k2_body(xb_ref, xT_ref, idx_ref):
    b = pl.program_id(0)
    xb = xb_ref[0]
    xT = xT_ref[0]
    d2 = jnp.dot(xb, xT, preferred_element_type=jnp.float32) * (-2.0)
    sqi = jnp.sum(xb * xb, axis=1, keepdims=True)
    sqj = jnp.sum(xT * xT, axis=0, keepdims=True)
    vals = d2 + sqi + sqj
    col = lax.broadcasted_iota(jnp.int32, (_R2, _N), 1)
    inf = jnp.float32(3e38)
    idxs = []
    for _ in range(_K):
        m = jnp.min(vals, axis=1, keepdims=True)
        am = jnp.min(jnp.where(vals <= m, col, _N), axis=1, keepdims=True)
        idxs.append(am)
        vals = jnp.where(col == am, inf, vals)
    idx_ref[0] = jnp.concatenate(idxs, axis=1) + b * _N


def _topk(xb, xT):
    return pl.pallas_call(
        _k2_body,
        grid=(_B, _N // _R2),
        in_specs=[
            pl.BlockSpec((1, _R2, 8), lambda b, i: (b, i, 0)),
            pl.BlockSpec((1, 8, _N), lambda b, i: (b, 0, 0)),
        ],
        out_specs=pl.BlockSpec((1, _R2, _K), lambda b, i: (b, i, 0)),
        out_shape=jax.ShapeDtypeStruct((_B, _N, _K), jnp.int32),
    )(xb, xT)


_NC, _NS = 2, 16
_NW = _NC * _NS
_PERW = _TOT // _NW
_CH = 128
_NCH = _PERW // _CH


def _gather_sc(T2, flat_idx):
    mesh = plsc.VectorSubcoreMesh(core_axis_name="c", subcore_axis_name="s")

    @functools.partial(
        pl.kernel, mesh=mesh,
        out_type=jax.ShapeDtypeStruct((_TOT, 384), jnp.float32),
        scratch_types=[
            pltpu.VMEM((_CH,), jnp.int32),
            pltpu.VMEM((_CH, 384), jnp.float32),
            pltpu.SemaphoreType.DMA,
        ],
    )
    def k3(T_hbm, idx_hbm, G_hbm, idx_v, rows_v, sem1):
        wid = lax.axis_index("s") * _NC + lax.axis_index("c")

        def body(c, carry):
            base = wid * _PERW + c * _CH
            pltpu.sync_copy(idx_hbm.at[pl.ds(base, _CH)], idx_v)
            pltpu.async_copy(T_hbm.at[idx_v], rows_v, sem1).wait()
            pltpu.sync_copy(rows_v, G_hbm.at[pl.ds(base, _CH)])
            return carry

        lax.fori_loop(0, _NCH, body, 0)

    return k3(T2, flat_idx)


def _k4_body(G_ref, xyz_ref, P_ref, res_ref, Wd_ref, bd_ref,
             Wg_ref, W2_ref, b2_ref, o_ref):
    S = G_ref[:, 0:128]
    A = G_ref[:, 128:256]
    xn = G_ref[:, 256:272]
    xc = xyz_ref[0]
    xcr = jnp.reshape(jnp.broadcast_to(xc[:, None, :], (_R4, _K, 16)), (_RK, 16))
    diff = xcr - xn
    delta = jnp.maximum(
        jnp.dot(diff, Wd_ref[...], preferred_element_type=jnp.float32) + bd_ref[...], 0.0)
    dg = jnp.dot(delta, Wg_ref[...], preferred_element_type=jnp.float32)
    Pc = P_ref[0]
    Pr = jnp.reshape(jnp.broadcast_to(Pc[:, None, :], (_R4, _K, _D)), (_RK, _D))
    gamma = Pr - S + dg
    m = jnp.max(gamma, axis=1, keepdims=True)
    e = jnp.exp(gamma - m)
    rho = e / jnp.sum(e, axis=1, keepdims=True)
    contrib = rho * (A + delta)
    agg = jnp.sum(jnp.reshape(contrib, (_R4, _K, _D)), axis=1)
    o_ref[0] = (jnp.dot(agg, W2_ref[...], preferred_element_type=jnp.float32)
                + b2_ref[...] + res_ref[0])


def _attention(G, xyzp, P, feat, Wdp, bd, Wg, W2, b2):
    nblk = _N // _R4
    wspec = pl.BlockSpec((128, 128), lambda b, i: (0, 0))
    bspec = pl.BlockSpec((1, 128), lambda b, i: (0, 0))
    return pl.pallas_call(
        _k4_body,
        grid=(_B, nblk),
        in_specs=[
            pl.BlockSpec((_RK, 384), lambda b, i: (b * nblk + i, 0)),
            pl.BlockSpec((1, _R4, 16), lambda b, i: (b, i, 0)),
            pl.BlockSpec((1, _R4, _D), lambda b, i: (b, i, 0)),
            pl.BlockSpec((1, _R4, _D), lambda b, i: (b, i, 0)),
            pl.BlockSpec((16, 128), lambda b, i: (0, 0)),
            bspec, wspec, wspec, bspec,
        ],
        out_specs=pl.BlockSpec((1, _R4, _D), lambda b, i: (b, i, 0)),
        out_shape=jax.ShapeDtypeStruct((_B, _N, _D), jnp.float32),
    )(G, xyzp, P, feat, Wdp, bd, Wg, W2, b2)


def kernel(points_xyz, points_features, W1, b1, Wg, bg, Wphi, bphi,
           Wpsi, bpsi, Wa, ba, Wd1, bd1, Wd2, bd2, W2, b2):
    Wpg = Wpsi @ Wg
    bS = (bpsi @ Wg).reshape(1, _D)
    Wpp = Wphi @ Wg
    bP = (bphi @ Wg + bg).reshape(1, _D)
    Wd = Wd1 @ Wd2
    bd = (bd1 @ Wd2 + bd2).reshape(1, _D)
    Wdp = jnp.zeros((16, _D), jnp.float32).at[0:3].set(Wd)

    xyzp = jnp.pad(points_xyz, ((0, 0), (0, 0), (0, 13)))
    xyzp128 = jnp.pad(points_xyz, ((0, 0), (0, 0), (0, 125)))
    xT = jnp.transpose(xyzp[..., :8], (0, 2, 1))

    T, P = _project(points_features, xyzp128, W1, b1.reshape(1, _D), Wa,
                    ba.reshape(1, _D), Wpg, bS, Wpp, bP)
    idx = _topk(xyzp[..., :8], xT)
    G = _gather_sc(T.reshape(_B * _N, 384), idx.reshape(_TOT))
    out_final = _attention(G, xyzp, P, points_features, Wdp, bd, Wg, W2,
                           b2.reshape(1, _D))
    return (points_xyz, out_final)
